# Initial kernel scaffold; baseline (speedup 1.0000x reference)
import jax
import jax.numpy as jnp
from jax.experimental import pallas as pl


def kernel(ctc_emissions, ctc_transition, ctc_bos, ctc_eos, targets):
    raise NotImplementedError("write your pallas kernel here")


# speedup vs baseline: 444.1618x; 444.1618x over previous
import jax
import jax.numpy as jnp
from jax.experimental import pallas as pl
from jax.experimental.pallas import tpu as pltpu

NEG = -1e30


def _lerp2v(c0, c1, c2, c3, u, v):
    a = c0 + u * (c1 - c0)
    b = c2 + u * (c3 - c2)
    return a + v * (b - a)


def _fb_kernel(em_ref, tgt_ref, aux_ref, trans_ref, bos_ref, eos_ref, out_ref):
    T = em_ref.shape[0]
    B, L = tgt_ref.shape

    tgt = tgt_ref[...]
    b0 = (tgt & 1).astype(jnp.float32)
    b1 = (tgt >> 1).astype(jnp.float32)

    tn = jnp.concatenate([tgt[:, 1:], tgt[:, :1]], axis=1)
    r0 = (tn & 1).astype(jnp.float32)
    r1 = (tn >> 1).astype(jnp.float32)

    w = []
    for i in range(4):
        a = trans_ref[i, 0] + r0 * (trans_ref[i, 1] - trans_ref[i, 0])
        b = trans_ref[i, 2] + r0 * (trans_ref[i, 3] - trans_ref[i, 2])
        w.append(a + r1 * (b - a))
    wa = w[0] + b0 * (w[1] - w[0])
    wb = w[2] + b0 * (w[3] - w[2])
    ty = wa + b1 * (wb - wa)

    em0 = em_ref[0]
    ent0 = em0[:, 0:4]

    u0 = b0[:, 0:1]
    v0 = b1[:, 0:1]
    e_first = _lerp2v(ent0[:, 0:1], ent0[:, 1:2], ent0[:, 2:3], ent0[:, 3:4], u0, v0)
    bos_g = (bos_ref[0, 0] + u0 * (bos_ref[0, 1] - bos_ref[0, 0])
             + v0 * ((bos_ref[0, 2] + u0 * (bos_ref[0, 3] - bos_ref[0, 2]))
                     - (bos_ref[0, 0] + u0 * (bos_ref[0, 1] - bos_ref[0, 0]))))
    first = bos_g + e_first
    lane = jax.lax.broadcasted_iota(jnp.int32, (B, L), 1)
    a_num0 = jnp.where(lane == 0, first, NEG)

    a_den0 = ent0 + aux_ref[0:1, 0:4]

    neg_col = jnp.full((B, 1), NEG, dtype=jnp.float32)

    def body(t, carry):
        a_den, a_num = carry
        em_t = em_ref[t]
        ent = em_t[:, 0:4]
        ext = em_t[:, 4:8]

        cols = []
        for j in range(4):
            vj = a_den + aux_ref[2 + j:3 + j, 0:4]
            m = jnp.max(vj, axis=1, keepdims=True)
            s = jnp.sum(jnp.exp(vj - m), axis=1, keepdims=True)
            cols.append(m + jnp.log(s))
        move_d = jnp.concatenate(cols, axis=1) + ent
        stay_d = a_den + ext
        mxd = jnp.maximum(move_d, stay_d)
        mnd = jnp.minimum(move_d, stay_d)
        a_den = mxd + jnp.log1p(jnp.exp(mnd - mxd))

        e_ent = _lerp2v(ent[:, 0:1], ent[:, 1:2], ent[:, 2:3], ent[:, 3:4], b0, b1)
        e_ext = _lerp2v(ext[:, 0:1], ext[:, 1:2], ext[:, 2:3], ext[:, 3:4], b0, b1)
        stay = a_num + e_ext
        tmp = a_num + ty
        shifted = jnp.concatenate([neg_col, tmp[:, :L - 1]], axis=1)
        move = shifted + e_ent
        mx = jnp.maximum(stay, move)
        mn = jnp.minimum(stay, move)
        a_num = mx + jnp.log1p(jnp.exp(mn - mx))
        return a_den, a_num

    a_den, a_num = jax.lax.fori_loop(1, T, body, (a_den0, a_num0))

    vd = a_den + aux_ref[1:2, 0:4]
    md = jnp.max(vd, axis=1, keepdims=True)
    sd = jnp.sum(jnp.exp(vd - md), axis=1, keepdims=True)
    logz_den = md + jnp.log(sd)

    ul = b0[:, L - 1:L]
    vl = b1[:, L - 1:L]
    ea = eos_ref[0, 0] + ul * (eos_ref[0, 1] - eos_ref[0, 0])
    eb = eos_ref[0, 2] + ul * (eos_ref[0, 3] - eos_ref[0, 2])
    eos_g = ea + vl * (eb - ea)
    logz_num = a_num[:, L - 1:L] + eos_g

    diff = logz_den - logz_num
    loss = jnp.sum(diff) * (1.0 / B)
    out_ref[...] = jnp.broadcast_to(loss, out_ref.shape)


def kernel(ctc_emissions, ctc_transition, ctc_bos, ctc_eos, targets):
    em = ctc_emissions.astype(jnp.float32)
    B, T, _ = em.shape
    em_tbc = jnp.moveaxis(em, 1, 0)

    trans = ctc_transition.astype(jnp.float32)
    bos = ctc_bos.astype(jnp.float32)
    eos = ctc_eos.astype(jnp.float32)

    aux = jnp.zeros((8, 128), jnp.float32)
    aux = aux.at[0, :4].set(bos)
    aux = aux.at[1, :4].set(eos)
    aux = aux.at[2:6, :4].set(trans.T)

    out = pl.pallas_call(
        _fb_kernel,
        out_shape=jax.ShapeDtypeStruct((8, 128), jnp.float32),
        in_specs=[
            pl.BlockSpec(memory_space=pltpu.VMEM),
            pl.BlockSpec(memory_space=pltpu.VMEM),
            pl.BlockSpec(memory_space=pltpu.VMEM),
            pl.BlockSpec(memory_space=pltpu.SMEM),
            pl.BlockSpec(memory_space=pltpu.SMEM),
            pl.BlockSpec(memory_space=pltpu.SMEM),
        ],
        out_specs=pl.BlockSpec(memory_space=pltpu.VMEM),
    )(em_tbc, targets.astype(jnp.int32), aux, trans, bos.reshape(1, 4),
      eos.reshape(1, 4))
    return out[0, 0]
